# baseline (speedup 1.0000x reference)
import functools

import ml_dtypes
import numpy as np

import jax
import jax.numpy as jnp
from jax import lax
from jax.experimental import pallas as pl
from jax.experimental.pallas import tpu as pltpu
from jax.experimental.pallas import tpu_sc as plsc

N = 4096
NC = 2
NS = 16
NW = NC * NS
NQ_SC = 2048
NQ_TC = N - NQ_SC
QPW = NQ_SC // NW
LANES = 16
Q32 = QPW // 32
BMQ = 128

BIG = np.float32(1e30)
BF16 = ml_dtypes.bfloat16

def _sqrt16(x):
    i = plsc.bitcast(x, jnp.int32)
    i = (i >> 1) + np.int32(0x1FBD1DF5)
    y = plsc.bitcast(i, jnp.float32)
    for _ in range(3):
        y = np.float32(0.5) * (y + x / y)
    return jnp.where(x > 0.0, y, np.float32(0.0))


def _knn_body(pts_t_hbm, out_hbm, pts_v, outv):
    wid = lax.axis_index("s") * NC + lax.axis_index("c")
    base = wid * QPW
    pltpu.sync_copy(pts_t_hbm, pts_v)

    zeros = jnp.zeros((LANES,), jnp.float32)
    fmt = plsc.PackFormat.INTERLEAVED

    qx, qy, qz = [], [], []
    for u in range(Q32):
        lo = pl.ds(base + u * 32, LANES)
        hi = pl.ds(base + u * 32 + LANES, LANES)
        qx.append(plsc.pack(pts_v[0, lo], pts_v[0, hi], format=fmt))
        qy.append(plsc.pack(pts_v[1, lo], pts_v[1, hi], format=fmt))
        qz.append(plsc.pack(pts_v[2, lo], pts_v[2, hi], format=fmt))

    def step(jv, carry):
        m1, m2, m3, m4 = (list(c) for c in carry)
        off = pl.multiple_of(jv * LANES, LANES)
        csl = pl.ds(off, LANES)
        cxv = pts_v[0, csl]
        cyv = pts_v[1, csl]
        czv = pts_v[2, csl]
        for l in range(LANES):
            cxs = jnp.broadcast_to(cxv[l], (LANES,))
            cys = jnp.broadcast_to(cyv[l], (LANES,))
            czs = jnp.broadcast_to(czv[l], (LANES,))
            cxb = plsc.pack(cxs, cxs, format=fmt)
            cyb = plsc.pack(cys, cys, format=fmt)
            czb = plsc.pack(czs, czs, format=fmt)
            for u in range(Q32):
                dx = qx[u] - cxb
                s = dx * dx
                dy = qy[u] - cyb
                s = s + dy * dy
                dz = qz[u] - czb
                s = s + dz * dz
                hi1 = jnp.maximum(m1[u], s)
                m1[u] = jnp.minimum(m1[u], s)
                hi2 = jnp.maximum(m2[u], hi1)
                m2[u] = jnp.minimum(m2[u], hi1)
                hi3 = jnp.maximum(m3[u], hi2)
                m3[u] = jnp.minimum(m3[u], hi2)
                m4[u] = jnp.minimum(m4[u], hi3)
        return m1, m2, m3, m4

    big16 = jnp.full((32,), 1e30, jnp.bfloat16)
    init = tuple([big16 for _ in range(Q32)] for _ in range(4))
    _, m2, m3, m4 = plsc.parallel_loop(
        0, N // LANES, step=1, unroll=2, carry=init)(step)

    third = np.float32(1.0 / 3.0)
    for u in range(Q32):
        h1 = plsc.unpack(m2[u], format=fmt)
        h2 = plsc.unpack(m3[u], format=fmt)
        h3 = plsc.unpack(m4[u], format=fmt)
        for half in range(2):
            mean = (_sqrt16(h1[half]) + _sqrt16(h2[half])
                    + _sqrt16(h3[half])) * third
            sc = jnp.maximum(mean, np.float32(1e-5)) * np.float32(0.001)
            dval = sc * sc
            sl = pl.ds(u * 32 + half * LANES, LANES)
            for k in range(9):
                outv[k, sl] = dval if k in (0, 4, 8) else zeros

    pltpu.sync_copy(outv, out_hbm.at[wid])


def _knn_sc(pts_t):
    mesh = plsc.VectorSubcoreMesh(
        core_axis_name="c", subcore_axis_name="s",
        num_cores=NC, num_subcores=NS)
    fn = functools.partial(
        pl.kernel,
        out_type=jax.ShapeDtypeStruct((NW, 9, QPW), jnp.float32),
        mesh=mesh,
        scratch_types=[
            pltpu.VMEM((3, N), jnp.float32),
            pltpu.VMEM((9, QPW), jnp.float32),
        ],
        compiler_params=pltpu.CompilerParams(
            needs_layout_passes=False, has_side_effects=False),
    )(_knn_body)
    return fn(pts_t)


def _tc_body(pts_ref, q_ref, out_ref):
    qoff = NQ_SC + pl.program_id(0) * BMQ
    row_iota = lax.broadcasted_iota(jnp.int32, (N, BMQ), 0)
    lane_iota = lax.broadcasted_iota(jnp.int32, (N, BMQ), 1)
    dist = jnp.zeros((N, BMQ), jnp.float32)
    for d in range(3):
        diff = pts_ref[:, d:d + 1] - q_ref[d:d + 1, :]
        dist = dist + diff * diff
    dist = jnp.where(row_iota == lane_iota + qoff, BIG, dist)
    m = []
    for t in range(3):
        mt = jnp.min(dist, axis=0, keepdims=True)
        m.append(mt)
        if t < 2:
            idx = jnp.min(jnp.where(dist == mt, row_iota, N),
                          axis=0, keepdims=True)
            dist = jnp.where(row_iota == idx, BIG, dist)
    third = np.float32(1.0 / 3.0)
    mean = (jnp.sqrt(m[0]) + jnp.sqrt(m[1]) + jnp.sqrt(m[2])) * third
    sc = jnp.maximum(mean, np.float32(1e-5)) * np.float32(0.001)
    dval = sc * sc
    zero = jnp.zeros((1, BMQ), jnp.float32)
    for k in range(9):
        out_ref[:, k, :] = dval if k in (0, 4, 8) else zero


def _knn_tc(points, q_t):
    grid = NQ_TC // BMQ
    return pl.pallas_call(
        _tc_body,
        grid=(grid,),
        in_specs=[
            pl.BlockSpec((N, 3), lambda i: (0, 0)),
            pl.BlockSpec((3, BMQ), lambda i: (0, i)),
        ],
        out_specs=pl.BlockSpec((1, 9, BMQ), lambda i: (i, 0, 0)),
        out_shape=jax.ShapeDtypeStruct((grid, 9, BMQ), jnp.float32),
        compiler_params=pltpu.CompilerParams(has_side_effects=False),
    )(points, q_t)


@jax.jit
def _knn(points, pts_t):
    sc_out = _knn_sc(pts_t)
    sc_cov = jnp.transpose(sc_out, (0, 2, 1)).reshape(NQ_SC, 3, 3)
    if NQ_TC == 0:
        return sc_cov
    tc_out = _knn_tc(points, pts_t[:, NQ_SC:])
    tc_cov = jnp.transpose(tc_out, (0, 2, 1)).reshape(NQ_TC, 3, 3)
    return jnp.concatenate([sc_cov, tc_cov], axis=0)


def kernel(points, colors):
    del colors
    pts_t = points.T
    return _knn(points, pts_t)


# speedup vs baseline: 1.0839x; 1.0839x over previous
import functools

import ml_dtypes
import numpy as np

import jax
import jax.numpy as jnp
from jax import lax
from jax.experimental import pallas as pl
from jax.experimental.pallas import tpu as pltpu
from jax.experimental.pallas import tpu_sc as plsc

N = 4096
NC = 2
NS = 16
NW = NC * NS
NQ_SC = 4096
NQ_TC = N - NQ_SC
QPW = NQ_SC // NW
LANES = 16
Q32 = QPW // 32
BMQ = 128

BIG = np.float32(1e30)
BF16 = ml_dtypes.bfloat16

def _sqrt16(x):
    i = plsc.bitcast(x, jnp.int32)
    i = (i >> 1) + np.int32(0x1FBD1DF5)
    y = plsc.bitcast(i, jnp.float32)
    for _ in range(3):
        y = np.float32(0.5) * (y + x / y)
    return jnp.where(x > 0.0, y, np.float32(0.0))


def _knn_body(pts_t_hbm, out_hbm, pts_v, outv):
    wid = lax.axis_index("s") * NC + lax.axis_index("c")
    base = wid * QPW
    pltpu.sync_copy(pts_t_hbm, pts_v)

    zeros = jnp.zeros((LANES,), jnp.float32)
    fmt = plsc.PackFormat.INTERLEAVED

    qx, qy, qz = [], [], []
    for u in range(Q32):
        lo = pl.ds(base + u * 32, LANES)
        hi = pl.ds(base + u * 32 + LANES, LANES)
        qx.append(plsc.pack(pts_v[0, lo], pts_v[0, hi], format=fmt))
        qy.append(plsc.pack(pts_v[1, lo], pts_v[1, hi], format=fmt))
        qz.append(plsc.pack(pts_v[2, lo], pts_v[2, hi], format=fmt))

    def step(jv, carry):
        m1, m2, m3, m4 = (list(c) for c in carry)
        off = pl.multiple_of(jv * LANES, LANES)
        csl = pl.ds(off, LANES)
        cxv = pts_v[0, csl]
        cyv = pts_v[1, csl]
        czv = pts_v[2, csl]
        for l in range(LANES):
            cxs = jnp.broadcast_to(cxv[l], (LANES,))
            cys = jnp.broadcast_to(cyv[l], (LANES,))
            czs = jnp.broadcast_to(czv[l], (LANES,))
            cxb = plsc.pack(cxs, cxs, format=fmt)
            cyb = plsc.pack(cys, cys, format=fmt)
            czb = plsc.pack(czs, czs, format=fmt)
            for u in range(Q32):
                dx = qx[u] - cxb
                s = dx * dx
                dy = qy[u] - cyb
                s = s + dy * dy
                dz = qz[u] - czb
                s = s + dz * dz
                hi1 = jnp.maximum(m1[u], s)
                m1[u] = jnp.minimum(m1[u], s)
                hi2 = jnp.maximum(m2[u], hi1)
                m2[u] = jnp.minimum(m2[u], hi1)
                hi3 = jnp.maximum(m3[u], hi2)
                m3[u] = jnp.minimum(m3[u], hi2)
                m4[u] = jnp.minimum(m4[u], hi3)
        return m1, m2, m3, m4

    big16 = jnp.full((32,), 1e30, jnp.bfloat16)
    init = tuple([big16 for _ in range(Q32)] for _ in range(4))
    _, m2, m3, m4 = plsc.parallel_loop(
        0, N // LANES, step=1, unroll=2, carry=init)(step)

    third = np.float32(1.0 / 3.0)
    for u in range(Q32):
        h1 = plsc.unpack(m2[u], format=fmt)
        h2 = plsc.unpack(m3[u], format=fmt)
        h3 = plsc.unpack(m4[u], format=fmt)
        for half in range(2):
            mean = (_sqrt16(h1[half]) + _sqrt16(h2[half])
                    + _sqrt16(h3[half])) * third
            sc = jnp.maximum(mean, np.float32(1e-5)) * np.float32(0.001)
            dval = sc * sc
            sl = pl.ds(u * 32 + half * LANES, LANES)
            for k in range(9):
                outv[k, sl] = dval if k in (0, 4, 8) else zeros

    pltpu.sync_copy(outv, out_hbm.at[wid])


def _knn_sc(pts_t):
    mesh = plsc.VectorSubcoreMesh(
        core_axis_name="c", subcore_axis_name="s",
        num_cores=NC, num_subcores=NS)
    fn = functools.partial(
        pl.kernel,
        out_type=jax.ShapeDtypeStruct((NW, 9, QPW), jnp.float32),
        mesh=mesh,
        scratch_types=[
            pltpu.VMEM((3, N), jnp.float32),
            pltpu.VMEM((9, QPW), jnp.float32),
        ],
        compiler_params=pltpu.CompilerParams(needs_layout_passes=False),
    )(_knn_body)
    return fn(pts_t)


def _tc_body(pts_ref, q_ref, out_ref):
    qoff = NQ_SC + pl.program_id(0) * BMQ
    row_iota = lax.broadcasted_iota(jnp.int32, (N, BMQ), 0)
    lane_iota = lax.broadcasted_iota(jnp.int32, (N, BMQ), 1)
    dist = jnp.zeros((N, BMQ), jnp.float32)
    for d in range(3):
        diff = pts_ref[:, d:d + 1] - q_ref[d:d + 1, :]
        dist = dist + diff * diff
    dist = jnp.where(row_iota == lane_iota + qoff, BIG, dist)
    m = []
    for t in range(3):
        mt = jnp.min(dist, axis=0, keepdims=True)
        m.append(mt)
        if t < 2:
            idx = jnp.min(jnp.where(dist == mt, row_iota, N),
                          axis=0, keepdims=True)
            dist = jnp.where(row_iota == idx, BIG, dist)
    third = np.float32(1.0 / 3.0)
    mean = (jnp.sqrt(m[0]) + jnp.sqrt(m[1]) + jnp.sqrt(m[2])) * third
    sc = jnp.maximum(mean, np.float32(1e-5)) * np.float32(0.001)
    dval = sc * sc
    zero = jnp.zeros((1, BMQ), jnp.float32)
    for k in range(9):
        out_ref[:, k, :] = dval if k in (0, 4, 8) else zero


def _knn_tc(points, q_t):
    grid = NQ_TC // BMQ
    return pl.pallas_call(
        _tc_body,
        grid=(grid,),
        in_specs=[
            pl.BlockSpec((N, 3), lambda i: (0, 0)),
            pl.BlockSpec((3, BMQ), lambda i: (0, i)),
        ],
        out_specs=pl.BlockSpec((1, 9, BMQ), lambda i: (i, 0, 0)),
        out_shape=jax.ShapeDtypeStruct((grid, 9, BMQ), jnp.float32),
        compiler_params=pltpu.CompilerParams(has_side_effects=False),
    )(points, q_t)


@jax.jit
def _knn(points, pts_t):
    sc_out = _knn_sc(pts_t)
    sc_cov = jnp.transpose(sc_out, (0, 2, 1)).reshape(NQ_SC, 3, 3)
    if NQ_TC == 0:
        return sc_cov
    tc_out = _knn_tc(points, pts_t[:, NQ_SC:])
    tc_cov = jnp.transpose(tc_out, (0, 2, 1)).reshape(NQ_TC, 3, 3)
    return jnp.concatenate([sc_cov, tc_cov], axis=0)


def kernel(points, colors):
    del colors
    pts_t = points.T
    return _knn(points, pts_t)
